# baseline (device time: 45489 ns/iter reference)
import jax
import jax.numpy as jnp
from jax import lax
from jax.experimental import pallas as pl
from jax.experimental.pallas import tpu as pltpu

N_DEV = 4


def kernel(dy, W):
    m, kdim = dy.shape
    n = W.shape[0]
    half = m // 4
    quar = m // 8

    def body(dy_ref, w_ref, out_ref, dyv, wv, wb,
             s1A, s1B, s2a, s2b, s3a, s3b, rA, rB,
             in_sems, sendA, recvA, sendB, recvB):
        my = lax.axis_index("i")
        bit0 = my & 1
        bit1 = (my >> 1) & 1
        p1 = my ^ 1
        p3 = my ^ 3

        ka = bit0 ^ bit1
        ma = bit0
        kb = bit1
        mb = bit0

        a_sh = (1 - ka) * half
        a_sa1 = a_sh + ma * quar
        a_sa2 = a_sh + (1 - ma) * quar
        a_da = ka * half + (1 - ma) * quar
        a_qa = ka * half + ma * quar
        b0 = 2 * half
        b_sh = b0 + (1 - kb) * half
        b_sb1 = b_sh + mb * quar
        b_sb2 = b_sh + (1 - mb) * quar
        b_db = b0 + kb * half + (1 - mb) * quar
        b_qb = b0 + kb * half + mb * quar

        def load(src, dst, i):
            c = pltpu.make_async_copy(src, dst, in_sems.at[i])
            c.start()
            return c

        cpw = load(w_ref, wv, 0)
        cp_sa = load(dy_ref.at[pl.ds(a_sh, half)], dyv.at[pl.ds(a_sh, half)], 1)
        cp_sb = load(dy_ref.at[pl.ds(b_sh, half)], dyv.at[pl.ds(b_sh, half)], 2)
        cp_da = load(dy_ref.at[pl.ds(a_da, quar)], dyv.at[pl.ds(a_da, quar)], 3)
        cp_db = load(dy_ref.at[pl.ds(b_db, quar)], dyv.at[pl.ds(b_db, quar)], 4)
        cp_qa = load(dy_ref.at[pl.ds(a_qa, quar)], dyv.at[pl.ds(a_qa, quar)], 5)
        cp_qb = load(dy_ref.at[pl.ds(b_qb, quar)], dyv.at[pl.ds(b_qb, quar)], 6)

        barrier_sem = pltpu.get_barrier_semaphore()
        for nbr in (p1, p3):
            pl.semaphore_signal(
                barrier_sem, inc=1,
                device_id=(nbr,), device_id_type=pl.DeviceIdType.MESH,
            )
        pl.semaphore_wait(barrier_sem, 2)

        def dot_rows(off, nrows):
            out_ref[pl.ds(off, nrows), :] = lax.dot_general(
                dyv[pl.ds(off, nrows), :].astype(jnp.bfloat16),
                wb[...],
                dimension_numbers=(((1,), (1,)), ((), ())),
                preferred_element_type=jnp.float32,
            )

        def xchg(src, dst, sems_s, sems_r, i, tgt):
            r = pltpu.make_async_remote_copy(
                src_ref=src, dst_ref=dst,
                send_sem=sems_s.at[i], recv_sem=sems_r.at[i],
                device_id=(tgt,), device_id_type=pl.DeviceIdType.MESH,
            )
            r.start()
            return r

        def reduce_into(off, rbuf, slot):
            cur = out_ref[pl.ds(off, quar), :]
            out_ref[pl.ds(off, quar), :] = cur + rbuf[slot].astype(jnp.float32)

        def final_store(off, rbuf, slot):
            out_ref[pl.ds(off, quar), :] = rbuf[slot].astype(jnp.float32)

        cpw.wait()
        wb[...] = wv[...].astype(jnp.bfloat16)
        cp_sa.wait()
        dot_rows(a_sh, half)
        s1A[...] = out_ref[pl.ds(a_sh, half), :].astype(jnp.bfloat16)
        m1a = xchg(s1A.at[pl.ds(ma * quar, quar)], rA.at[0], sendA, recvA, 0, p1)
        m1b = xchg(s1A.at[pl.ds((1 - ma) * quar, quar)], rA.at[1],
                   sendA, recvA, 1, p1)
        cp_sb.wait()
        dot_rows(b_sh, half)
        s1B[...] = out_ref[pl.ds(b_sh, half), :].astype(jnp.bfloat16)
        n1a = xchg(s1B.at[pl.ds(mb * quar, quar)], rB.at[0], sendB, recvB, 0, p3)
        n1b = xchg(s1B.at[pl.ds((1 - mb) * quar, quar)], rB.at[1],
                   sendB, recvB, 1, p3)

        cp_da.wait()
        dot_rows(a_da, quar)
        cp_db.wait()
        dot_rows(b_db, quar)

        m1a.wait()
        reduce_into(a_da, rA, 0)
        s2a[...] = out_ref[pl.ds(a_da, quar), :].astype(jnp.bfloat16)
        m2 = xchg(s2a, rA.at[2], sendA, recvA, 2, p3)
        n1a.wait()
        reduce_into(b_db, rB, 0)
        s2b[...] = out_ref[pl.ds(b_db, quar), :].astype(jnp.bfloat16)
        n2 = xchg(s2b, rB.at[2], sendB, recvB, 2, p1)

        cp_qa.wait()
        dot_rows(a_qa, quar)
        cp_qb.wait()
        dot_rows(b_qb, quar)
        m1b.wait()
        reduce_into(a_qa, rA, 1)
        n1b.wait()
        reduce_into(b_qb, rB, 1)

        m2.wait()
        reduce_into(a_qa, rA, 2)
        s3a[...] = out_ref[pl.ds(a_qa, quar), :].astype(jnp.bfloat16)
        m3 = xchg(s3a, rA.at[3], sendA, recvA, 3, p3)
        m4a = xchg(s3a, rA.at[4], sendA, recvA, 4, p1)
        n2.wait()
        reduce_into(b_qb, rB, 2)
        s3b[...] = out_ref[pl.ds(b_qb, quar), :].astype(jnp.bfloat16)
        n3 = xchg(s3b, rB.at[3], sendB, recvB, 3, p1)
        n4a = xchg(s3b, rB.at[4], sendB, recvB, 4, p3)

        m3.wait()
        m4b = xchg(rA.at[3], rA.at[5], sendA, recvA, 5, p1)
        final_store(a_da, rA, 3)
        n3.wait()
        n4b = xchg(rB.at[3], rB.at[5], sendB, recvB, 5, p3)
        final_store(b_db, rB, 3)

        m4a.wait()
        final_store(a_sa2, rA, 4)
        n4a.wait()
        final_store(b_sb2, rB, 4)
        m4b.wait()
        final_store(a_sa1, rA, 5)
        n4b.wait()
        final_store(b_sb1, rB, 5)

    return pl.pallas_call(
        body,
        out_shape=jax.ShapeDtypeStruct((m, n), jnp.float32),
        in_specs=[
            pl.BlockSpec(memory_space=pl.ANY),
            pl.BlockSpec(memory_space=pl.ANY),
        ],
        out_specs=pl.BlockSpec(memory_space=pltpu.VMEM),
        scratch_shapes=[
            pltpu.VMEM((m, kdim), jnp.float32),
            pltpu.VMEM((n, kdim), jnp.float32),
            pltpu.VMEM((n, kdim), jnp.bfloat16),
            pltpu.VMEM((half, n), jnp.bfloat16),
            pltpu.VMEM((half, n), jnp.bfloat16),
            pltpu.VMEM((quar, n), jnp.bfloat16),
            pltpu.VMEM((quar, n), jnp.bfloat16),
            pltpu.VMEM((quar, n), jnp.bfloat16),
            pltpu.VMEM((quar, n), jnp.bfloat16),
            pltpu.VMEM((6, quar, n), jnp.bfloat16),
            pltpu.VMEM((6, quar, n), jnp.bfloat16),
            pltpu.SemaphoreType.DMA((7,)),
            pltpu.SemaphoreType.DMA((6,)),
            pltpu.SemaphoreType.DMA((6,)),
            pltpu.SemaphoreType.DMA((6,)),
            pltpu.SemaphoreType.DMA((6,)),
        ],
        compiler_params=pltpu.CompilerParams(
            collective_id=0,
            vmem_limit_bytes=100 * 1024 * 1024,
        ),
    )(dy, W)
